# Initial kernel scaffold; baseline (speedup 1.0000x reference)
import jax
import jax.numpy as jnp
from jax.experimental import pallas as pl


def kernel(x, cu_seqlens, weight, bias):
    raise NotImplementedError("write your pallas kernel here")


# speedup vs baseline: 2.8288x; 2.8288x over previous
import functools

import jax
import jax.numpy as jnp
from jax import lax
from jax.experimental import pallas as pl
from jax.experimental.pallas import tpu as pltpu
from jax.experimental.pallas import tpu_sc as plsc

TOTAL = 32768
DIM = 128
NSEG = 16
EPS = 1e-5
NC = 2
NS = 16
NW = NC * NS
RPW = TOTAL // NW
CHUNK = 128
NCHUNK = RPW // CHUNK
LANES = 16
VPR = DIM // LANES

_mesh = plsc.VectorSubcoreMesh(
    core_axis_name="c", subcore_axis_name="s", num_cores=NC, num_subcores=NS
)


def _worker_id():
    return lax.axis_index("s") * NC + lax.axis_index("c")


def _lane_sum_bcast(v):
    idx = lax.iota(jnp.int32, LANES)
    dnums = lax.GatherDimensionNumbers(
        offset_dims=(), collapsed_slice_dims=(0,), start_index_map=(0,)
    )
    for sh in (8, 4, 2, 1):
        perm = jnp.remainder(idx + sh, LANES)
        v = v + lax.gather(v, perm[:, None], dnums, (1,),
                           mode=lax.GatherScatterMode.PROMISE_IN_BOUNDS)
    return v


@functools.partial(
    pl.kernel,
    out_type=jax.ShapeDtypeStruct((NW, 2, NSEG), jnp.float32),
    mesh=_mesh,
    scratch_types=[
        pltpu.VMEM((NSEG,), jnp.int32),
        pltpu.VMEM((NSEG,), jnp.int32),
        pltpu.VMEM((CHUNK, DIM), jnp.float32),
        pltpu.VMEM((2, NSEG, LANES), jnp.float32),
        pltpu.VMEM((2, NSEG), jnp.float32),
    ],
)
def _stats_kernel(x_hbm, cu0_hbm, cu1_hbm, out_hbm, a_v, b_v, buf, acc_v, p_v):
    wid = _worker_id()
    lo = wid * RPW
    pltpu.sync_copy(cu0_hbm, a_v)
    pltpu.sync_copy(cu1_hbm, b_v)
    a = jnp.clip(a_v[...], lo, lo + RPW)
    b = jnp.clip(b_v[...], lo, lo + RPW)

    zero = jnp.zeros((LANES,), jnp.float32)
    for t in range(2):
        for g in range(NSEG):
            acc_v[t, g, :] = zero

    def chunk_body(c, carry):
        base = lo + c * CHUNK
        pltpu.sync_copy(x_hbm.at[pl.ds(base, CHUNK)], buf)
        for g in range(NSEG):
            s = jnp.maximum(a[g], base) - base
            e = jnp.minimum(b[g], base + CHUNK) - base
            acc = acc_v[0, g, :]
            accq = acc_v[1, g, :]

            def row_body(r, rc):
                ac, aq = rc
                for j in range(VPR):
                    v = buf[r, pl.ds(j * LANES, LANES)]
                    ac = ac + v
                    aq = aq + v * v
                return (ac, aq)

            acc, accq = lax.fori_loop(s, e, row_body, (acc, accq))
            acc_v[0, g, :] = acc
            acc_v[1, g, :] = accq
        return carry

    lax.fori_loop(0, NCHUNK, chunk_body, 0)

    lane = lax.iota(jnp.int32, NSEG)
    p0 = jnp.zeros((NSEG,), jnp.float32)
    p1 = jnp.zeros((NSEG,), jnp.float32)
    for g in range(NSEG):
        p0 = jnp.where(lane == g, _lane_sum_bcast(acc_v[0, g, :]), p0)
        p1 = jnp.where(lane == g, _lane_sum_bcast(acc_v[1, g, :]), p1)
    p_v[0, :] = p0
    p_v[1, :] = p1
    pltpu.sync_copy(p_v, out_hbm.at[wid])


@functools.partial(
    pl.kernel,
    out_type=jax.ShapeDtypeStruct((TOTAL, DIM), jnp.float32),
    mesh=_mesh,
    scratch_types=[
        pltpu.VMEM((NSEG,), jnp.int32),
        pltpu.VMEM((NSEG,), jnp.int32),
        pltpu.VMEM((NW, 2, NSEG), jnp.float32),
        pltpu.VMEM((1, DIM), jnp.float32),
        pltpu.VMEM((1, DIM), jnp.float32),
        pltpu.VMEM((CHUNK, DIM), jnp.float32),
        pltpu.VMEM((CHUNK, DIM), jnp.float32),
    ],
)
def _norm_kernel(x_hbm, cu0_hbm, cu1_hbm, part_hbm, w_hbm, bias_hbm, out_hbm,
                 a_v, b_v, p_v, w_v, bb_v, ibuf, obuf):
    wid = _worker_id()
    lo = wid * RPW
    pltpu.sync_copy(cu0_hbm, a_v)
    pltpu.sync_copy(cu1_hbm, b_v)
    pltpu.sync_copy(part_hbm, p_v)
    pltpu.sync_copy(w_hbm, w_v)
    pltpu.sync_copy(bias_hbm, bb_v)

    s1 = jnp.zeros((NSEG,), jnp.float32)
    s2 = jnp.zeros((NSEG,), jnp.float32)
    for w in range(NW):
        s1 = s1 + p_v[w, 0, :]
        s2 = s2 + p_v[w, 1, :]

    cu0 = a_v[...]
    cu1 = b_v[...]
    denom = jnp.maximum((cu1 - cu0).astype(jnp.float32) * DIM, float(DIM))
    mean = s1 / denom
    var = jnp.maximum(s2 / denom - mean * mean, 0.0) + EPS
    seed_i = jnp.int32(0x5F3759DF) - (lax.bitcast_convert_type(var, jnp.int32) >> 1)
    y = lax.bitcast_convert_type(seed_i, jnp.float32)
    for _ in range(3):
        y = y * (1.5 - 0.5 * var * y * y)
    a = jnp.clip(cu0, lo, lo + RPW)
    b = jnp.clip(cu1, lo, lo + RPW)

    wvec = [w_v[0, pl.ds(j * LANES, LANES)] for j in range(VPR)]
    bvec = [bb_v[0, pl.ds(j * LANES, LANES)] for j in range(VPR)]

    def chunk_body(c, carry):
        base = lo + c * CHUNK
        pltpu.sync_copy(x_hbm.at[pl.ds(base, CHUNK)], ibuf)
        for g in range(NSEG):
            s = jnp.maximum(a[g], base) - base
            e = jnp.minimum(b[g], base + CHUNK) - base
            mg = mean[g]
            rg = y[g]
            scales = [rg * wvec[j] for j in range(VPR)]
            shifts = [bvec[j] - mg * scales[j] for j in range(VPR)]

            def row_body(r, rc):
                for j in range(VPR):
                    obuf[r, pl.ds(j * LANES, LANES)] = (
                        ibuf[r, pl.ds(j * LANES, LANES)] * scales[j] + shifts[j]
                    )
                return rc

            lax.fori_loop(s, e, row_body, 0)
        pltpu.sync_copy(obuf, out_hbm.at[pl.ds(base, CHUNK)])
        return carry

    lax.fori_loop(0, NCHUNK, chunk_body, 0)


def kernel(x, cu_seqlens, weight, bias):
    cu = cu_seqlens.astype(jnp.int32)
    cu0 = cu[:NSEG]
    cu1 = cu[1:NSEG + 1]
    partials = _stats_kernel(x, cu0, cu1)
    return _norm_kernel(x, cu0, cu1, partials, weight, bias)


# speedup vs baseline: 3.1671x; 1.1196x over previous
import functools

import jax
import jax.numpy as jnp
from jax import lax
from jax.experimental import pallas as pl
from jax.experimental.pallas import tpu as pltpu
from jax.experimental.pallas import tpu_sc as plsc

TOTAL = 32768
DIM = 128
NSEG = 16
EPS = 1e-5
NC = 2
NS = 16
NW = NC * NS
RPW = TOTAL // NW
CHUNK = 128
NCHUNK = RPW // CHUNK
LANES = 16
VPR = DIM // LANES

_mesh = plsc.VectorSubcoreMesh(
    core_axis_name="c", subcore_axis_name="s", num_cores=NC, num_subcores=NS
)


def _worker_id():
    return lax.axis_index("s") * NC + lax.axis_index("c")


def _lane_sum_bcast(v):
    idx = lax.iota(jnp.int32, LANES)
    dnums = lax.GatherDimensionNumbers(
        offset_dims=(), collapsed_slice_dims=(0,), start_index_map=(0,)
    )
    for sh in (8, 4, 2, 1):
        perm = jnp.remainder(idx + sh, LANES)
        v = v + lax.gather(v, perm[:, None], dnums, (1,),
                           mode=lax.GatherScatterMode.PROMISE_IN_BOUNDS)
    return v


@functools.partial(
    pl.kernel,
    out_type=jax.ShapeDtypeStruct((NW, 2, NSEG), jnp.float32),
    mesh=_mesh,
    scratch_types=[
        pltpu.VMEM((NSEG,), jnp.int32),
        pltpu.VMEM((NSEG,), jnp.int32),
        pltpu.VMEM((2, CHUNK, DIM), jnp.float32),
        pltpu.VMEM((2, NSEG, LANES), jnp.float32),
        pltpu.VMEM((2, NSEG), jnp.float32),
        pltpu.SemaphoreType.DMA,
        pltpu.SemaphoreType.DMA,
    ],
)
def _stats_kernel(x_hbm, cu0_hbm, cu1_hbm, out_hbm, a_v, b_v, ibuf, acc_v,
                  p_v, si0, si1):
    wid = _worker_id()
    lo = wid * RPW
    si = (si0, si1)

    def start_in(c, par):
        pltpu.async_copy(x_hbm.at[pl.ds(lo + c * CHUNK, CHUNK)],
                         ibuf.at[par], si[par])

    def wait_in(par):
        pltpu.make_async_copy(x_hbm.at[pl.ds(0, CHUNK)],
                              ibuf.at[par], si[par]).wait()

    start_in(0, 0)
    pltpu.sync_copy(cu0_hbm, a_v)
    pltpu.sync_copy(cu1_hbm, b_v)
    a = jnp.clip(a_v[...], lo, lo + RPW)
    b = jnp.clip(b_v[...], lo, lo + RPW)

    zero = jnp.zeros((LANES,), jnp.float32)
    for t in range(2):
        for g in range(NSEG):
            acc_v[t, g, :] = zero

    def chunk2_body(c2, carry):
        for par in range(2):
            c = 2 * c2 + par

            @pl.when(c + 1 < NCHUNK)
            def _():
                start_in(c + 1, 1 - par)

            wait_in(par)
            base = lo + c * CHUNK
            for g in range(NSEG):
                s = jnp.maximum(a[g], base) - base
                e = jnp.minimum(b[g], base + CHUNK) - base
                acc = acc_v[0, g, :]
                accq = acc_v[1, g, :]

                def row_body(r, rc):
                    ac, aq = rc
                    for j in range(VPR):
                        v = ibuf[par, r, pl.ds(j * LANES, LANES)]
                        ac = ac + v
                        aq = aq + v * v
                    return (ac, aq)

                acc, accq = lax.fori_loop(s, e, row_body, (acc, accq))
                acc_v[0, g, :] = acc
                acc_v[1, g, :] = accq
        return carry

    lax.fori_loop(0, NCHUNK // 2, chunk2_body, 0)

    lane = lax.iota(jnp.int32, NSEG)
    p0 = jnp.zeros((NSEG,), jnp.float32)
    p1 = jnp.zeros((NSEG,), jnp.float32)
    for g in range(NSEG):
        p0 = jnp.where(lane == g, _lane_sum_bcast(acc_v[0, g, :]), p0)
        p1 = jnp.where(lane == g, _lane_sum_bcast(acc_v[1, g, :]), p1)
    p_v[0, :] = p0
    p_v[1, :] = p1
    pltpu.sync_copy(p_v, out_hbm.at[wid])


@functools.partial(
    pl.kernel,
    out_type=jax.ShapeDtypeStruct((TOTAL, DIM), jnp.float32),
    mesh=_mesh,
    scratch_types=[
        pltpu.VMEM((NSEG,), jnp.int32),
        pltpu.VMEM((NSEG,), jnp.int32),
        pltpu.VMEM((NW, 2, NSEG), jnp.float32),
        pltpu.VMEM((1, DIM), jnp.float32),
        pltpu.VMEM((1, DIM), jnp.float32),
        pltpu.VMEM((2, CHUNK, DIM), jnp.float32),
        pltpu.VMEM((2, CHUNK, DIM), jnp.float32),
        pltpu.SemaphoreType.DMA,
        pltpu.SemaphoreType.DMA,
        pltpu.SemaphoreType.DMA,
        pltpu.SemaphoreType.DMA,
    ],
)
def _norm_kernel(x_hbm, cu0_hbm, cu1_hbm, part_hbm, w_hbm, bias_hbm, out_hbm,
                 a_v, b_v, p_v, w_v, bb_v, ibuf, obuf, si0, si1, so0, so1):
    wid = _worker_id()
    lo = wid * RPW
    si = (si0, si1)
    so = (so0, so1)

    def start_in(c, par):
        pltpu.async_copy(x_hbm.at[pl.ds(lo + c * CHUNK, CHUNK)],
                         ibuf.at[par], si[par])

    def wait_in(par):
        pltpu.make_async_copy(x_hbm.at[pl.ds(0, CHUNK)],
                              ibuf.at[par], si[par]).wait()

    def start_out(c, par):
        pltpu.async_copy(obuf.at[par],
                         out_hbm.at[pl.ds(lo + c * CHUNK, CHUNK)], so[par])

    def wait_out(par):
        pltpu.make_async_copy(obuf.at[par],
                              out_hbm.at[pl.ds(0, CHUNK)], so[par]).wait()

    start_in(0, 0)
    pltpu.sync_copy(cu0_hbm, a_v)
    pltpu.sync_copy(cu1_hbm, b_v)
    pltpu.sync_copy(part_hbm, p_v)
    pltpu.sync_copy(w_hbm, w_v)
    pltpu.sync_copy(bias_hbm, bb_v)

    s1 = jnp.zeros((NSEG,), jnp.float32)
    s2 = jnp.zeros((NSEG,), jnp.float32)
    for w in range(NW):
        s1 = s1 + p_v[w, 0, :]
        s2 = s2 + p_v[w, 1, :]

    cu0 = a_v[...]
    cu1 = b_v[...]
    denom = jnp.maximum((cu1 - cu0).astype(jnp.float32) * DIM, float(DIM))
    mean = s1 / denom
    var = jnp.maximum(s2 / denom - mean * mean, 0.0) + EPS
    seed_i = jnp.int32(0x5F3759DF) - (lax.bitcast_convert_type(var, jnp.int32) >> 1)
    y = lax.bitcast_convert_type(seed_i, jnp.float32)
    for _ in range(3):
        y = y * (1.5 - 0.5 * var * y * y)
    a = jnp.clip(cu0, lo, lo + RPW)
    b = jnp.clip(cu1, lo, lo + RPW)

    wvec = [w_v[0, pl.ds(j * LANES, LANES)] for j in range(VPR)]
    bvec = [bb_v[0, pl.ds(j * LANES, LANES)] for j in range(VPR)]

    def chunk2_body(c2, carry):
        for par in range(2):
            c = 2 * c2 + par

            @pl.when(c + 1 < NCHUNK)
            def _():
                start_in(c + 1, 1 - par)

            wait_in(par)

            @pl.when(c >= 2)
            def _():
                wait_out(par)

            base = lo + c * CHUNK
            for g in range(NSEG):
                s = jnp.maximum(a[g], base) - base
                e = jnp.minimum(b[g], base + CHUNK) - base
                mg = mean[g]
                rg = y[g]
                scales = [rg * wvec[j] for j in range(VPR)]
                shifts = [bvec[j] - mg * scales[j] for j in range(VPR)]

                def row_body(r, rc):
                    for j in range(VPR):
                        obuf[par, r, pl.ds(j * LANES, LANES)] = (
                            ibuf[par, r, pl.ds(j * LANES, LANES)] * scales[j]
                            + shifts[j]
                        )
                    return rc

                lax.fori_loop(s, e, row_body, 0)
            start_out(c, par)
        return carry

    lax.fori_loop(0, NCHUNK // 2, chunk2_body, 0)
    wait_out(0)
    wait_out(1)


def kernel(x, cu_seqlens, weight, bias):
    cu = cu_seqlens.astype(jnp.int32)
    cu0 = cu[:NSEG]
    cu1 = cu[1:NSEG + 1]
    partials = _stats_kernel(x, cu0, cu1)
    return _norm_kernel(x, cu0, cu1, partials, weight, bias)


# speedup vs baseline: 4.6978x; 1.4833x over previous
import functools

import jax
import jax.numpy as jnp
from jax import lax
from jax.experimental import pallas as pl
from jax.experimental.pallas import tpu as pltpu
from jax.experimental.pallas import tpu_sc as plsc

TOTAL = 32768
DIM = 128
NSEG = 16
EPS = 1e-5
NC = 2
NS = 16
NW = NC * NS
RPW = TOTAL // NW
CHUNK = 128
NCHUNK = RPW // CHUNK
LANES = 16
VPR = DIM // LANES

_mesh = plsc.VectorSubcoreMesh(
    core_axis_name="c", subcore_axis_name="s", num_cores=NC, num_subcores=NS
)


def _worker_id():
    return lax.axis_index("s") * NC + lax.axis_index("c")


def _lane_sum_bcast(v):
    idx = lax.iota(jnp.int32, LANES)
    dnums = lax.GatherDimensionNumbers(
        offset_dims=(), collapsed_slice_dims=(0,), start_index_map=(0,)
    )
    for sh in (8, 4, 2, 1):
        perm = jnp.remainder(idx + sh, LANES)
        v = v + lax.gather(v, perm[:, None], dnums, (1,),
                           mode=lax.GatherScatterMode.PROMISE_IN_BOUNDS)
    return v


@functools.partial(
    pl.kernel,
    out_type=jax.ShapeDtypeStruct((NW, 2, NSEG), jnp.float32),
    mesh=_mesh,
    scratch_types=[
        pltpu.VMEM((NSEG,), jnp.int32),
        pltpu.VMEM((NSEG,), jnp.int32),
        pltpu.VMEM((2, CHUNK, DIM), jnp.float32),
        pltpu.VMEM((2, NSEG, LANES), jnp.float32),
        pltpu.VMEM((2, NSEG), jnp.float32),
        pltpu.SemaphoreType.DMA,
        pltpu.SemaphoreType.DMA,
    ],
)
def _stats_kernel(x_hbm, cu0_hbm, cu1_hbm, out_hbm, a_v, b_v, ibuf, acc_v,
                  p_v, si0, si1):
    wid = _worker_id()
    lo = wid * RPW
    si = (si0, si1)

    def start_in(c, par):
        pltpu.async_copy(x_hbm.at[pl.ds(lo + c * CHUNK, CHUNK)],
                         ibuf.at[par], si[par])

    def wait_in(par):
        pltpu.make_async_copy(x_hbm.at[pl.ds(0, CHUNK)],
                              ibuf.at[par], si[par]).wait()

    start_in(0, 0)
    pltpu.sync_copy(cu0_hbm, a_v)
    pltpu.sync_copy(cu1_hbm, b_v)
    a = jnp.clip(a_v[...], lo, lo + RPW)
    b = jnp.clip(b_v[...], lo, lo + RPW)

    zero = jnp.zeros((LANES,), jnp.float32)
    for t in range(2):
        for g in range(NSEG):
            acc_v[t, g, :] = zero

    def chunk2_body(c2, carry):
        for par in range(2):
            c = 2 * c2 + par

            @pl.when(c + 1 < NCHUNK)
            def _():
                start_in(c + 1, 1 - par)

            wait_in(par)
            base = lo + c * CHUNK
            cover = jnp.logical_and(a <= base, b >= base + CHUNK)
            covf = jnp.where(cover, 1.0, 0.0).astype(jnp.float32)
            pop = _lane_sum_bcast(covf)

            @pl.when(pop[0] > 0.5)
            def _():
                zero16 = jnp.zeros((LANES,), jnp.float32)

                def row_body(r, rc):
                    ac, aq = rc
                    vs = [ibuf[par, r, pl.ds(j * LANES, LANES)]
                          for j in range(VPR)]
                    sq = [v * v for v in vs]
                    t = ((vs[0] + vs[1]) + (vs[2] + vs[3])) + \
                        ((vs[4] + vs[5]) + (vs[6] + vs[7]))
                    tq = ((sq[0] + sq[1]) + (sq[2] + sq[3])) + \
                         ((sq[4] + sq[5]) + (sq[6] + sq[7]))
                    return (ac + t, aq + tq)

                accf, accqf = plsc.parallel_loop(
                    0, CHUNK, step=1, unroll=2, carry=(zero16, zero16)
                )(row_body)
                for g in range(NSEG):
                    cg = covf[g]
                    acc_v[0, g, :] = acc_v[0, g, :] + cg * accf
                    acc_v[1, g, :] = acc_v[1, g, :] + cg * accqf

            @pl.when(pop[0] < 0.5)
            def _():
                for g in range(NSEG):
                    s = jnp.maximum(a[g], base) - base
                    e = jnp.minimum(b[g], base + CHUNK) - base
                    acc = acc_v[0, g, :]
                    accq = acc_v[1, g, :]

                    def row_body(r, rc):
                        ac, aq = rc
                        for j in range(VPR):
                            v = ibuf[par, r, pl.ds(j * LANES, LANES)]
                            ac = ac + v
                            aq = aq + v * v
                        return (ac, aq)

                    acc, accq = lax.fori_loop(s, e, row_body, (acc, accq))
                    acc_v[0, g, :] = acc
                    acc_v[1, g, :] = accq
        return carry

    lax.fori_loop(0, NCHUNK // 2, chunk2_body, 0)

    lane = lax.iota(jnp.int32, NSEG)
    p0 = jnp.zeros((NSEG,), jnp.float32)
    p1 = jnp.zeros((NSEG,), jnp.float32)
    for g in range(NSEG):
        p0 = jnp.where(lane == g, _lane_sum_bcast(acc_v[0, g, :]), p0)
        p1 = jnp.where(lane == g, _lane_sum_bcast(acc_v[1, g, :]), p1)
    p_v[0, :] = p0
    p_v[1, :] = p1
    pltpu.sync_copy(p_v, out_hbm.at[wid])


@functools.partial(
    pl.kernel,
    out_type=jax.ShapeDtypeStruct((TOTAL, DIM), jnp.float32),
    mesh=_mesh,
    scratch_types=[
        pltpu.VMEM((NSEG,), jnp.int32),
        pltpu.VMEM((NSEG,), jnp.int32),
        pltpu.VMEM((NW, 2, NSEG), jnp.float32),
        pltpu.VMEM((1, DIM), jnp.float32),
        pltpu.VMEM((1, DIM), jnp.float32),
        pltpu.VMEM((2, CHUNK, DIM), jnp.float32),
        pltpu.VMEM((2, CHUNK, DIM), jnp.float32),
        pltpu.SemaphoreType.DMA,
        pltpu.SemaphoreType.DMA,
        pltpu.SemaphoreType.DMA,
        pltpu.SemaphoreType.DMA,
    ],
)
def _norm_kernel(x_hbm, cu0_hbm, cu1_hbm, part_hbm, w_hbm, bias_hbm, out_hbm,
                 a_v, b_v, p_v, w_v, bb_v, ibuf, obuf, si0, si1, so0, so1):
    wid = _worker_id()
    lo = wid * RPW
    si = (si0, si1)
    so = (so0, so1)

    def start_in(c, par):
        pltpu.async_copy(x_hbm.at[pl.ds(lo + c * CHUNK, CHUNK)],
                         ibuf.at[par], si[par])

    def wait_in(par):
        pltpu.make_async_copy(x_hbm.at[pl.ds(0, CHUNK)],
                              ibuf.at[par], si[par]).wait()

    def start_out(c, par):
        pltpu.async_copy(obuf.at[par],
                         out_hbm.at[pl.ds(lo + c * CHUNK, CHUNK)], so[par])

    def wait_out(par):
        pltpu.make_async_copy(obuf.at[par],
                              out_hbm.at[pl.ds(0, CHUNK)], so[par]).wait()

    start_in(0, 0)
    pltpu.sync_copy(cu0_hbm, a_v)
    pltpu.sync_copy(cu1_hbm, b_v)
    pltpu.sync_copy(part_hbm, p_v)
    pltpu.sync_copy(w_hbm, w_v)
    pltpu.sync_copy(bias_hbm, bb_v)

    s1 = jnp.zeros((NSEG,), jnp.float32)
    s2 = jnp.zeros((NSEG,), jnp.float32)
    for w in range(NW):
        s1 = s1 + p_v[w, 0, :]
        s2 = s2 + p_v[w, 1, :]

    cu0 = a_v[...]
    cu1 = b_v[...]
    denom = jnp.maximum((cu1 - cu0).astype(jnp.float32) * DIM, float(DIM))
    mean = s1 / denom
    var = jnp.maximum(s2 / denom - mean * mean, 0.0) + EPS
    seed_i = jnp.int32(0x5F3759DF) - (lax.bitcast_convert_type(var, jnp.int32) >> 1)
    y = lax.bitcast_convert_type(seed_i, jnp.float32)
    for _ in range(3):
        y = y * (1.5 - 0.5 * var * y * y)
    a = jnp.clip(cu0, lo, lo + RPW)
    b = jnp.clip(cu1, lo, lo + RPW)

    wvec = [w_v[0, pl.ds(j * LANES, LANES)] for j in range(VPR)]
    bvec = [bb_v[0, pl.ds(j * LANES, LANES)] for j in range(VPR)]

    def chunk2_body(c2, carry):
        for par in range(2):
            c = 2 * c2 + par

            @pl.when(c + 1 < NCHUNK)
            def _():
                start_in(c + 1, 1 - par)

            wait_in(par)

            @pl.when(c >= 2)
            def _():
                wait_out(par)

            base = lo + c * CHUNK
            cover = jnp.logical_and(a <= base, b >= base + CHUNK)
            pop = _lane_sum_bcast(jnp.where(cover, 1.0, 0.0).astype(jnp.float32))

            @pl.when(pop[0] > 0.5)
            def _():
                mgv = _lane_sum_bcast(jnp.where(cover, mean, 0.0))
                rgv = _lane_sum_bcast(jnp.where(cover, y, 0.0))
                scales = [rgv * wvec[j] for j in range(VPR)]
                shifts = [bvec[j] - mgv * scales[j] for j in range(VPR)]

                def row_body(r):
                    for j in range(VPR):
                        obuf[par, r, pl.ds(j * LANES, LANES)] = (
                            ibuf[par, r, pl.ds(j * LANES, LANES)] * scales[j]
                            + shifts[j]
                        )

                plsc.parallel_loop(0, CHUNK, step=1, unroll=2)(row_body)

            @pl.when(pop[0] < 0.5)
            def _():
                for g in range(NSEG):
                    s = jnp.maximum(a[g], base) - base
                    e = jnp.minimum(b[g], base + CHUNK) - base
                    mg = mean[g]
                    rg = y[g]
                    scales = [rg * wvec[j] for j in range(VPR)]
                    shifts = [bvec[j] - mg * scales[j] for j in range(VPR)]

                    def row_body(r, rc):
                        for j in range(VPR):
                            obuf[par, r, pl.ds(j * LANES, LANES)] = (
                                ibuf[par, r, pl.ds(j * LANES, LANES)]
                                * scales[j] + shifts[j]
                            )
                        return rc

                    lax.fori_loop(s, e, row_body, 0)
            start_out(c, par)
        return carry

    lax.fori_loop(0, NCHUNK // 2, chunk2_body, 0)
    wait_out(0)
    wait_out(1)


def kernel(x, cu_seqlens, weight, bias):
    cu = cu_seqlens.astype(jnp.int32)
    cu0 = cu[:NSEG]
    cu1 = cu[1:NSEG + 1]
    partials = _stats_kernel(x, cu0, cu1)
    return _norm_kernel(x, cu0, cu1, partials, weight, bias)


# speedup vs baseline: 4.9472x; 1.0531x over previous
import functools

import jax
import jax.numpy as jnp
from jax import lax
from jax.experimental import pallas as pl
from jax.experimental.pallas import tpu as pltpu
from jax.experimental.pallas import tpu_sc as plsc

TOTAL = 32768
DIM = 128
NSEG = 16
EPS = 1e-5
NC = 2
NS = 16
NW = NC * NS
RPW = TOTAL // NW
CHUNK = 128
NCHUNK = RPW // CHUNK
LANES = 16
VPR = DIM // LANES

_mesh = plsc.VectorSubcoreMesh(
    core_axis_name="c", subcore_axis_name="s", num_cores=NC, num_subcores=NS
)


def _lane_sum_bcast(v):
    idx = lax.iota(jnp.int32, LANES)
    dnums = lax.GatherDimensionNumbers(
        offset_dims=(), collapsed_slice_dims=(0,), start_index_map=(0,)
    )
    for sh in (8, 4, 2, 1):
        perm = jnp.remainder(idx + sh, LANES)
        v = v + lax.gather(v, perm[:, None], dnums, (1,),
                           mode=lax.GatherScatterMode.PROMISE_IN_BOUNDS)
    return v


@functools.partial(
    pl.kernel,
    out_type=(
        jax.ShapeDtypeStruct((TOTAL, DIM), jnp.float32),
        jax.ShapeDtypeStruct((NW, 3, NSEG), jnp.float32),
    ),
    mesh=_mesh,
    scratch_types=[
        pltpu.VMEM((NSEG,), jnp.int32),
        pltpu.VMEM((NSEG,), jnp.int32),
        pltpu.VMEM((1, DIM), jnp.float32),
        pltpu.VMEM((1, DIM), jnp.float32),
        pltpu.VMEM((1, DIM), jnp.float32),
        pltpu.VMEM((2, CHUNK, DIM), jnp.float32),
        pltpu.VMEM((2, CHUNK, DIM), jnp.float32),
        pltpu.VMEM((2, NSEG, LANES), jnp.float32),
        pltpu.VMEM((1, 3, NSEG), jnp.float32),
        pltpu.VMEM((NW, 3, NSEG), jnp.float32),
        pltpu.VMEM((LANES,), jnp.float32),
        pltpu.SemaphoreType.DMA,
        pltpu.SemaphoreType.DMA,
        pltpu.SemaphoreType.DMA,
        pltpu.SemaphoreType.DMA,
    ],
)
def _fused_kernel(x_hbm, cu0_hbm, cu1_hbm, w_hbm, bias_hbm, y_hbm, comm_hbm,
                  a_v, b_v, w_v, bb_v, r0_v, ibuf, obuf, acc_v, p_v, cm_v,
                  done_v, si0, si1, so0, so1):
    wid = lax.axis_index("s") * NC + lax.axis_index("c")
    lo = wid * RPW
    si = (si0, si1)
    so = (so0, so1)

    def start_in(c, par):
        pltpu.async_copy(x_hbm.at[pl.ds(lo + c * CHUNK, CHUNK)],
                         ibuf.at[par], si[par])

    def wait_in(par):
        pltpu.make_async_copy(x_hbm.at[pl.ds(0, CHUNK)],
                              ibuf.at[par], si[par]).wait()

    def start_out(c, par):
        pltpu.async_copy(obuf.at[par],
                         y_hbm.at[pl.ds(lo + c * CHUNK, CHUNK)], so[par])

    def wait_out(par):
        pltpu.make_async_copy(obuf.at[par],
                              y_hbm.at[pl.ds(0, CHUNK)], so[par]).wait()

    start_in(0, 0)
    pltpu.sync_copy(cu0_hbm, a_v)
    pltpu.sync_copy(cu1_hbm, b_v)
    pltpu.sync_copy(w_hbm, w_v)
    pltpu.sync_copy(bias_hbm, bb_v)
    pltpu.sync_copy(x_hbm.at[pl.ds(0, 1)], r0_v)
    tag_i = lax.bitcast_convert_type(r0_v[0, pl.ds(0, LANES)], jnp.int32)

    cu0 = a_v[...]
    cu1 = b_v[...]
    a = jnp.clip(cu0, lo, lo + RPW)
    b = jnp.clip(cu1, lo, lo + RPW)

    zero = jnp.zeros((LANES,), jnp.float32)
    for t in range(2):
        for g in range(NSEG):
            acc_v[t, g, :] = zero

    def stats_chunk2(c2, carry):
        for par in range(2):
            c = 2 * c2 + par

            @pl.when(c + 1 < NCHUNK)
            def _():
                start_in(c + 1, 1 - par)

            wait_in(par)
            base = lo + c * CHUNK
            cover = jnp.logical_and(a <= base, b >= base + CHUNK)
            covf = jnp.where(cover, 1.0, 0.0).astype(jnp.float32)
            pop = _lane_sum_bcast(covf)

            @pl.when(pop[0] > 0.5)
            def _():
                zero16 = jnp.zeros((LANES,), jnp.float32)

                def row_body(r, rc):
                    ac, aq = rc
                    vs = [ibuf[par, r, pl.ds(j * LANES, LANES)]
                          for j in range(VPR)]
                    sq = [v * v for v in vs]
                    t = ((vs[0] + vs[1]) + (vs[2] + vs[3])) + \
                        ((vs[4] + vs[5]) + (vs[6] + vs[7]))
                    tq = ((sq[0] + sq[1]) + (sq[2] + sq[3])) + \
                         ((sq[4] + sq[5]) + (sq[6] + sq[7]))
                    return (ac + t, aq + tq)

                accf, accqf = plsc.parallel_loop(
                    0, CHUNK, step=1, unroll=4, carry=(zero16, zero16)
                )(row_body)
                for g in range(NSEG):
                    cg = covf[g]
                    acc_v[0, g, :] = acc_v[0, g, :] + cg * accf
                    acc_v[1, g, :] = acc_v[1, g, :] + cg * accqf

            @pl.when(pop[0] < 0.5)
            def _():
                for g in range(NSEG):
                    s = jnp.maximum(a[g], base) - base
                    e = jnp.minimum(b[g], base + CHUNK) - base
                    acc = acc_v[0, g, :]
                    accq = acc_v[1, g, :]

                    def row_body(r, rc):
                        ac, aq = rc
                        for j in range(VPR):
                            v = ibuf[par, r, pl.ds(j * LANES, LANES)]
                            ac = ac + v
                            aq = aq + v * v
                        return (ac, aq)

                    acc, accq = lax.fori_loop(s, e, row_body, (acc, accq))
                    acc_v[0, g, :] = acc
                    acc_v[1, g, :] = accq
        return carry

    lax.fori_loop(0, NCHUNK // 2, stats_chunk2, 0)

    lane = lax.iota(jnp.int32, NSEG)
    p0 = jnp.zeros((NSEG,), jnp.float32)
    p1 = jnp.zeros((NSEG,), jnp.float32)
    for g in range(NSEG):
        p0 = jnp.where(lane == g, _lane_sum_bcast(acc_v[0, g, :]), p0)
        p1 = jnp.where(lane == g, _lane_sum_bcast(acc_v[1, g, :]), p1)
    p_v[0, 0, :] = p0
    p_v[0, 1, :] = p1
    p_v[0, 2, :] = r0_v[0, pl.ds(0, LANES)]
    pltpu.sync_copy(p_v.at[:, pl.ds(0, 2)],
                    comm_hbm.at[pl.ds(wid, 1), pl.ds(0, 2)])
    pltpu.sync_copy(p_v.at[:, pl.ds(2, 1)],
                    comm_hbm.at[pl.ds(wid, 1), pl.ds(2, 1)])

    start_in(0, 0)

    done_v[...] = jnp.zeros((LANES,), jnp.float32)

    def poll_once():
        pltpu.sync_copy(comm_hbm, cm_v)
        ok = jnp.zeros((LANES,), jnp.float32)
        for w in range(NW):
            fi = lax.bitcast_convert_type(cm_v[w, 2, :], jnp.int32)
            ok = ok + jnp.where(fi == tag_i, 1.0, 0.0)
        tot = _lane_sum_bcast(ok)

        @pl.when(tot[0] > float(NW * LANES) - 0.5)
        def _():
            done_v[...] = jnp.ones((LANES,), jnp.float32)

    def poll_outer(i, carry):
        @pl.when(done_v[...][0] < 0.5)
        def _():
            def poll_inner(k, c2):
                @pl.when(done_v[...][0] < 0.5)
                def _():
                    poll_once()
                return c2

            lax.fori_loop(0, 32, poll_inner, 0)
        return carry

    lax.fori_loop(0, 32, poll_outer, 0)
    poison = jnp.where(done_v[...] > 0.5, 1.0, jnp.float32(jnp.nan))
    pltpu.sync_copy(comm_hbm, cm_v)

    s1 = jnp.zeros((NSEG,), jnp.float32)
    s2 = jnp.zeros((NSEG,), jnp.float32)
    for w in range(NW):
        s1 = s1 + cm_v[w, 0, :]
        s2 = s2 + cm_v[w, 1, :]

    denom = jnp.maximum((cu1 - cu0).astype(jnp.float32) * DIM, float(DIM))
    mean = s1 / denom
    var = jnp.maximum(s2 / denom - mean * mean, 0.0) + EPS
    seed_i = jnp.int32(0x5F3759DF) - (lax.bitcast_convert_type(var, jnp.int32) >> 1)
    y = lax.bitcast_convert_type(seed_i, jnp.float32)
    for _ in range(3):
        y = y * (1.5 - 0.5 * var * y * y)
    y = y * poison

    wvec = [w_v[0, pl.ds(j * LANES, LANES)] for j in range(VPR)]
    bvec = [bb_v[0, pl.ds(j * LANES, LANES)] for j in range(VPR)]

    def norm_chunk2(c2, carry):
        for par in range(2):
            c = 2 * c2 + par

            @pl.when(c + 1 < NCHUNK)
            def _():
                start_in(c + 1, 1 - par)

            wait_in(par)

            @pl.when(c >= 2)
            def _():
                wait_out(par)

            base = lo + c * CHUNK
            cover = jnp.logical_and(a <= base, b >= base + CHUNK)
            pop = _lane_sum_bcast(jnp.where(cover, 1.0, 0.0).astype(jnp.float32))

            @pl.when(pop[0] > 0.5)
            def _():
                mgv = _lane_sum_bcast(jnp.where(cover, mean, 0.0))
                rgv = _lane_sum_bcast(jnp.where(cover, y, 0.0))
                scales = [rgv * wvec[j] for j in range(VPR)]
                shifts = [bvec[j] - mgv * scales[j] for j in range(VPR)]

                def row_body(r):
                    for j in range(VPR):
                        obuf[par, r, pl.ds(j * LANES, LANES)] = (
                            ibuf[par, r, pl.ds(j * LANES, LANES)] * scales[j]
                            + shifts[j]
                        )

                plsc.parallel_loop(0, CHUNK, step=1, unroll=4)(row_body)

            @pl.when(pop[0] < 0.5)
            def _():
                for g in range(NSEG):
                    s = jnp.maximum(a[g], base) - base
                    e = jnp.minimum(b[g], base + CHUNK) - base
                    mg = mean[g]
                    rg = y[g]
                    scales = [rg * wvec[j] for j in range(VPR)]
                    shifts = [bvec[j] - mg * scales[j] for j in range(VPR)]

                    def row_body(r, rc):
                        for j in range(VPR):
                            obuf[par, r, pl.ds(j * LANES, LANES)] = (
                                ibuf[par, r, pl.ds(j * LANES, LANES)]
                                * scales[j] + shifts[j]
                            )
                        return rc

                    lax.fori_loop(s, e, row_body, 0)
            start_out(c, par)
        return carry

    lax.fori_loop(0, NCHUNK // 2, norm_chunk2, 0)
    wait_out(0)
    wait_out(1)


def kernel(x, cu_seqlens, weight, bias):
    cu = cu_seqlens.astype(jnp.int32)
    cu0 = cu[:NSEG]
    cu1 = cu[1:NSEG + 1]
    out, _ = _fused_kernel(x, cu0, cu1, weight, bias)
    return out
